# baseline (speedup 1.0000x reference)
import functools

import jax
import jax.numpy as jnp
from jax import lax
from jax.experimental import pallas as pl
from jax.experimental.pallas import tpu as pltpu
from jax.experimental.pallas import tpu_sc as plsc

D = 64
NI = 4096
NJ = 200
NW = 32
IB = 128
JB = 8
NTJ = NJ // JB
NTI = NI // IB

_mesh = plsc.VectorSubcoreMesh(core_axis_name="c", subcore_axis_name="s")


@functools.partial(
    pl.kernel,
    mesh=_mesh,
    out_type=jax.ShapeDtypeStruct((NJ, D // 8, NTI, 8, IB), jnp.float32),
    compiler_params=pltpu.CompilerParams(
        use_tc_tiling_on_sc=False, needs_layout_passes=False),
    scratch_types=[
        pltpu.VMEM((JB, IB), jnp.int32),
        pltpu.VMEM((IB, IB), jnp.float32),
        pltpu.VMEM((IB, IB), jnp.float32),
        pltpu.VMEM((D, IB), jnp.float32),
        pltpu.VMEM((D, IB), jnp.float32),
        pltpu.SemaphoreType.DMA,
        pltpu.SemaphoreType.DMA,
        pltpu.SemaphoreType.DMA,
        pltpu.SemaphoreType.DMA,
    ],
)
def _emb_lookup(xv_hbm, tp_hbm, out_hbm, idx_v, rb0, rb1, ob0, ob1,
                gs0, gs1, ws0, ws1):
    wid = lax.axis_index("s") * 2 + lax.axis_index("c")
    rb = (rb0, rb1)
    ob = (ob0, ob1)
    gs = (gs0, gs1)
    ws = (ws0, ws1)
    lanes = lax.iota(jnp.int32, 16)

    rows_g = [lanes + (16 * g) for g in range(8)]

    def transpose_block(src, dst):
        def d_body(d0, carry):
            dbase = d0 * 8
            cbase = jnp.full((16,), dbase, jnp.int32)
            for dd in range(8):
                cols = cbase + dd
                for g in range(8):
                    v = plsc.load_gather(src, [rows_g[g], cols])
                    dst[dbase + dd, pl.ds(16 * g, 16)] = v
            return carry
        lax.fori_loop(0, D // 8, d_body, 0)

    def tile_body(tj, carry):
        pltpu.sync_copy(xv_hbm.at[tj, wid], idx_v)
        copies = [None, None]
        writes = [None, None]
        copies[0] = pltpu.async_copy(tp_hbm.at[idx_v.at[0]], rb[0], gs[0])
        for jj in range(JB):
            cur = jj % 2
            nxt = 1 - cur
            if jj + 1 < JB:
                copies[nxt] = pltpu.async_copy(
                    tp_hbm.at[idx_v.at[jj + 1]], rb[nxt], gs[nxt])
            copies[cur].wait()
            if jj >= 2:
                for w in writes[cur]:
                    w.wait()
            transpose_block(rb[cur], ob[cur])
            jabs = tj * JB + jj
            writes[cur] = [
                pltpu.async_copy(
                    ob[cur].at[pl.ds(8 * tk, 8)],
                    out_hbm.at[jabs, tk, wid],
                    ws[cur])
                for tk in range(D // 8)
            ]
        for wl in writes:
            for w in wl:
                w.wait()
        return carry

    lax.fori_loop(0, NTJ, tile_body, 0)


def kernel(x, table):
    xv = x.T.reshape(NTJ, JB, NTI, IB).transpose(0, 2, 1, 3)
    tp = jnp.pad(table, ((0, 0), (0, IB - D)))
    o5 = _emb_lookup(xv, tp)
    out_t = o5.transpose(0, 1, 3, 2, 4).reshape(NJ, D, NI)
    return out_t.transpose(2, 0, 1)


# speedup vs baseline: 1.4255x; 1.4255x over previous
import functools

import jax
import jax.numpy as jnp
from jax import lax
from jax.experimental import pallas as pl
from jax.experimental.pallas import tpu as pltpu
from jax.experimental.pallas import tpu_sc as plsc

D = 64
NI = 4096
NJ = 200
NW = 32
IB = 128
JB = 8
NTJ = NJ // JB
NTI = NI // IB

_mesh = plsc.VectorSubcoreMesh(core_axis_name="c", subcore_axis_name="s")


@functools.partial(
    pl.kernel,
    mesh=_mesh,
    out_type=jax.ShapeDtypeStruct((NJ, D // 8, NTI, 8, IB), jnp.float32),
    compiler_params=pltpu.CompilerParams(
        use_tc_tiling_on_sc=False, needs_layout_passes=False),
    scratch_types=[
        pltpu.VMEM((JB, IB), jnp.int32),
        pltpu.VMEM((IB, IB), jnp.float32),
        pltpu.VMEM((IB, IB), jnp.float32),
        pltpu.VMEM((D, IB), jnp.float32),
        pltpu.VMEM((D, IB), jnp.float32),
        pltpu.SemaphoreType.DMA,
        pltpu.SemaphoreType.DMA,
        pltpu.SemaphoreType.DMA,
        pltpu.SemaphoreType.DMA,
    ],
)
def _emb_lookup(xv_hbm, tp_hbm, out_hbm, idx_v, rb0, rb1, ob0, ob1,
                gs0, gs1, ws0, ws1):
    wid = lax.axis_index("s") * 2 + lax.axis_index("c")
    rb = (rb0, rb1)
    ob = (ob0, ob1)
    gs = (gs0, gs1)
    ws = (ws0, ws1)
    lanes = lax.iota(jnp.int32, 16)

    rows_g = [lanes + (16 * g) for g in range(8)]

    def transpose_block(src, dst):
        @plsc.parallel_loop(0, D, unroll=8)
        def d_body(d):
            cols = jnp.full((16,), d, jnp.int32)
            for g in range(8):
                v = plsc.load_gather(src, [rows_g[g], cols])
                dst[d, pl.ds(16 * g, 16)] = v

    def tile_body(tj, carry):
        pltpu.sync_copy(xv_hbm.at[tj, wid], idx_v)
        copies = [None, None]
        writes = [None, None]
        copies[0] = pltpu.async_copy(tp_hbm.at[idx_v.at[0]], rb[0], gs[0])
        for jj in range(JB):
            cur = jj % 2
            nxt = 1 - cur
            if jj + 1 < JB:
                copies[nxt] = pltpu.async_copy(
                    tp_hbm.at[idx_v.at[jj + 1]], rb[nxt], gs[nxt])
            copies[cur].wait()
            if jj >= 2:
                for w in writes[cur]:
                    w.wait()
            transpose_block(rb[cur], ob[cur])
            jabs = tj * JB + jj
            writes[cur] = [
                pltpu.async_copy(
                    ob[cur].at[pl.ds(8 * tk, 8)],
                    out_hbm.at[jabs, tk, wid],
                    ws[cur])
                for tk in range(D // 8)
            ]
        for wl in writes:
            for w in wl:
                w.wait()
        return carry

    lax.fori_loop(0, NTJ, tile_body, 0)


def kernel(x, table):
    xv = x.T.reshape(NTJ, JB, NTI, IB).transpose(0, 2, 1, 3)
    tp = jnp.pad(table, ((0, 0), (0, IB - D)))
    o5 = _emb_lookup(xv, tp)
    out_t = o5.transpose(0, 1, 3, 2, 4).reshape(NJ, D, NI)
    return out_t.transpose(2, 0, 1)


# speedup vs baseline: 2.2423x; 1.5731x over previous
import functools

import jax
import jax.numpy as jnp
from jax import lax
from jax.experimental import pallas as pl
from jax.experimental.pallas import tpu as pltpu
from jax.experimental.pallas import tpu_sc as plsc

D = 64
NI = 4096
NJ = 200
NW = 32
IB = 128
JB = 8
NTJ = NJ // JB
NTI = NI // IB

_mesh = plsc.VectorSubcoreMesh(core_axis_name="c", subcore_axis_name="s")


@functools.partial(
    pl.kernel,
    mesh=_mesh,
    out_type=jax.ShapeDtypeStruct((NJ, D // 8, NTI, 8, IB), jnp.float32),
    compiler_params=pltpu.CompilerParams(
        use_tc_tiling_on_sc=False, needs_layout_passes=False),
    scratch_types=[
        pltpu.VMEM((JB, IB), jnp.int32),
        pltpu.VMEM((IB, IB), jnp.float32),
        pltpu.VMEM((IB, IB), jnp.float32),
        pltpu.VMEM((D, IB), jnp.float32),
        pltpu.VMEM((D, IB), jnp.float32),
        pltpu.SemaphoreType.DMA,
        pltpu.SemaphoreType.DMA,
        pltpu.SemaphoreType.DMA,
        pltpu.SemaphoreType.DMA,
    ],
)
def _emb_lookup(xv_hbm, tp_hbm, out_hbm, idx_v, rb0, rb1, ob0, ob1,
                gs0, gs1, ws0, ws1):
    wid = lax.axis_index("s") * 2 + lax.axis_index("c")
    rb = (rb0, rb1)
    ob = (ob0, ob1)
    gs = (gs0, gs1)
    ws = (ws0, ws1)
    lanes = lax.iota(jnp.int32, 16)

    rows_g = [lanes + (16 * g) for g in range(8)]

    def transpose_block(src, dst):
        @plsc.parallel_loop(0, D, unroll=8)
        def d_body(d):
            diag = (jnp.full((16,), d, jnp.int32) + lanes) & (D - 1)
            for g in range(8):
                v = plsc.load_gather(src, [rows_g[g], diag])
                plsc.store_scatter(dst, [diag, rows_g[g]], v)

    def tile_body(tj, carry):
        pltpu.sync_copy(xv_hbm.at[tj, wid], idx_v)
        copies = [None, None]
        writes = [None, None]
        copies[0] = pltpu.async_copy(tp_hbm.at[idx_v.at[0]], rb[0], gs[0])
        for jj in range(JB):
            cur = jj % 2
            nxt = 1 - cur
            if jj + 1 < JB:
                copies[nxt] = pltpu.async_copy(
                    tp_hbm.at[idx_v.at[jj + 1]], rb[nxt], gs[nxt])
            copies[cur].wait()
            if jj >= 2:
                for w in writes[cur]:
                    w.wait()
            transpose_block(rb[cur], ob[cur])
            jabs = tj * JB + jj
            writes[cur] = [
                pltpu.async_copy(
                    ob[cur].at[pl.ds(8 * tk, 8)],
                    out_hbm.at[jabs, tk, wid],
                    ws[cur])
                for tk in range(D // 8)
            ]
        for wl in writes:
            for w in wl:
                w.wait()
        return carry

    lax.fori_loop(0, NTJ, tile_body, 0)


def kernel(x, table):
    xv = x.T.reshape(NTJ, JB, NTI, IB).transpose(0, 2, 1, 3)
    tp = jnp.pad(table, ((0, 0), (0, IB - D)))
    o5 = _emb_lookup(xv, tp)
    out_t = o5.transpose(0, 1, 3, 2, 4).reshape(NJ, D, NI)
    return out_t.transpose(2, 0, 1)


# speedup vs baseline: 3.4982x; 1.5601x over previous
import functools

import jax
import jax.numpy as jnp
from jax import lax
from jax.experimental import pallas as pl
from jax.experimental.pallas import tpu as pltpu
from jax.experimental.pallas import tpu_sc as plsc

D = 64
NI = 4096
NJ = 200
NW = 32
IB = 128
JB = 8
NTJ = NJ // JB
NTI = NI // IB

_mesh = plsc.VectorSubcoreMesh(core_axis_name="c", subcore_axis_name="s")

NVB = (1000000 // IB)
VB_BASE = NVB // NW
VB_EXTRA = NVB % NW
NPAIR = 1000000 * D // IB


@functools.partial(
    pl.kernel,
    mesh=_mesh,
    out_type=jax.ShapeDtypeStruct((NPAIR, IB), jnp.float32),
    compiler_params=pltpu.CompilerParams(
        use_tc_tiling_on_sc=True, needs_layout_passes=False),
    scratch_types=[
        pltpu.VMEM((D, IB), jnp.float32),
        pltpu.VMEM((D, IB), jnp.float32),
        pltpu.VMEM((D, IB), jnp.float32),
        pltpu.VMEM((D, IB), jnp.float32),
        pltpu.VMEM((IB // 4, IB), jnp.float32),
        pltpu.SemaphoreType.DMA,
        pltpu.SemaphoreType.DMA,
        pltpu.SemaphoreType.DMA,
        pltpu.SemaphoreType.DMA,
    ],
)
def _relayout(tt_hbm, patch_hbm, tp_hbm, sb0, sb1, db0, db1, pbuf,
              rs0, rs1, ws0, ws1):
    wid = lax.axis_index("s") * 2 + lax.axis_index("c")
    cnt = VB_BASE + jnp.where(wid < VB_EXTRA, 1, 0)
    start = wid * VB_BASE + jnp.minimum(wid, VB_EXTRA)
    sb = (sb0, sb1)
    db = (db0, db1)
    rs = (rs0, rs1)
    ws = (ws0, ws1)
    lanes = lax.iota(jnp.int32, 16)
    vl_g = [lanes + 16 * g for g in range(8)]
    q_g = [(lanes + 16 * g) >> 1 for g in range(8)]
    pc_g = [((lanes + 16 * g) & 1) * D for g in range(8)]

    def transpose_vb(src, dst):
        @plsc.parallel_loop(0, D, unroll=8)
        def d_body(d0):
            diag = (jnp.full((16,), d0, jnp.int32) + lanes) & (D - 1)
            for g in range(8):
                v = plsc.load_gather(src, [diag, vl_g[g]])
                plsc.store_scatter(dst, [q_g[g], pc_g[g] + diag], v)

    def issue_read(i, h):
        tv = start + i
        return pltpu.async_copy(
            tt_hbm.at[pl.ds(0, D), pl.ds(pl.multiple_of(tv * IB, IB), IB)],
            sb[h], rs[h])

    @pl.when(cnt > 0)
    def _prime():
        issue_read(0, 0)

    def pair_body(t, carry):
        for h in range(2):
            i = t * 2 + h

            @pl.when(i < cnt)
            def _do():
                @pl.when(i + 1 < cnt)
                def _ahead():
                    issue_read(i + 1, 1 - h)
                pltpu.make_async_copy(tt_hbm.at[pl.ds(0, D),
                                                pl.ds(0, IB)],
                                      sb[h], rs[h]).wait()

                @pl.when(i >= 2)
                def _drain():
                    pltpu.make_async_copy(db[h],
                                          tp_hbm.at[pl.ds(0, D)],
                                          ws[h]).wait()
                transpose_vb(sb[h], db[h])
                prow = pl.multiple_of((start + i) * (D // 2) * 2, 8)
                pltpu.async_copy(db[h], tp_hbm.at[pl.ds(prow, D)], ws[h])
        return carry

    lax.fori_loop(0, (VB_BASE + 2) // 2, pair_body, 0)
    for h in range(2):
        @pl.when(cnt > h)
        def _final_drain():
            pltpu.make_async_copy(db[h], tp_hbm.at[pl.ds(0, D)],
                                  ws[h]).wait()

    @pl.when(wid == NW - 1)
    def _tail_patch():
        pltpu.sync_copy(patch_hbm, pbuf)
        pltpu.sync_copy(pbuf, tp_hbm.at[pl.ds(NPAIR - IB // 4, IB // 4)])


@functools.partial(
    pl.kernel,
    mesh=_mesh,
    out_type=jax.ShapeDtypeStruct((NJ, D // 8, NTI, 8, IB), jnp.float32),
    compiler_params=pltpu.CompilerParams(
        use_tc_tiling_on_sc=False, needs_layout_passes=False),
    scratch_types=[
        pltpu.VMEM((JB, IB), jnp.int32),
        pltpu.VMEM((JB, IB), jnp.int32),
        pltpu.VMEM((JB, IB), jnp.int32),
        pltpu.VMEM((IB, IB), jnp.float32),
        pltpu.VMEM((IB, IB), jnp.float32),
        pltpu.VMEM((D, IB), jnp.float32),
        pltpu.VMEM((D, IB), jnp.float32),
        pltpu.SemaphoreType.DMA,
        pltpu.SemaphoreType.DMA,
        pltpu.SemaphoreType.DMA,
        pltpu.SemaphoreType.DMA,
    ],
)
def _emb_lookup(xv_hbm, tp_hbm, out_hbm, idx_v, pair_v, pofs_v,
                rb0, rb1, ob0, ob1, gs0, gs1, ws0, ws1):
    wid = lax.axis_index("s") * 2 + lax.axis_index("c")
    rb = (rb0, rb1)
    ob = (ob0, ob1)
    gs = (gs0, gs1)
    ws = (ws0, ws1)
    lanes = lax.iota(jnp.int32, 16)

    rows_g = [lanes + (16 * g) for g in range(8)]

    def transpose_block(src, dst, pofs_row):
        pv = [pofs_row[pl.ds(16 * g, 16)] for g in range(8)]

        @plsc.parallel_loop(0, D, unroll=8)
        def d_body(d):
            diag = (jnp.full((16,), d, jnp.int32) + lanes) & (D - 1)
            for g in range(8):
                v = plsc.load_gather(src, [rows_g[g], diag + pv[g]])
                plsc.store_scatter(dst, [diag, rows_g[g]], v)

    def tile_body(tj, carry):
        pltpu.sync_copy(xv_hbm.at[tj, wid], idx_v)

        @plsc.parallel_loop(0, JB * IB, step=16, unroll=8)
        def split_body(k):
            r = k // IB
            c = k % IB
            v = idx_v[r, pl.ds(c, 16)]
            pair_v[r, pl.ds(c, 16)] = v >> 1
            pofs_v[r, pl.ds(c, 16)] = (v & 1) << 6

        copies = [None, None]
        writes = [None, None]
        copies[0] = pltpu.async_copy(tp_hbm.at[pair_v.at[0]], rb[0], gs[0])
        for jj in range(JB):
            cur = jj % 2
            nxt = 1 - cur
            if jj + 1 < JB:
                copies[nxt] = pltpu.async_copy(
                    tp_hbm.at[pair_v.at[jj + 1]], rb[nxt], gs[nxt])
            copies[cur].wait()
            if jj >= 2:
                for w in writes[cur]:
                    w.wait()
            transpose_block(rb[cur], ob[cur], pofs_v.at[jj])
            jabs = tj * JB + jj
            writes[cur] = [
                pltpu.async_copy(
                    ob[cur].at[pl.ds(8 * tk, 8)],
                    out_hbm.at[jabs, tk, wid],
                    ws[cur])
                for tk in range(D // 8)
            ]
        for wl in writes:
            for w in wl:
                w.wait()
        return carry

    lax.fori_loop(0, NTJ, tile_body, 0)


def kernel(x, table):
    xv = x.T.reshape(NTJ, JB, NTI, IB).transpose(0, 2, 1, 3)
    tt = table.T
    patch = table[NVB * IB:].reshape(IB // 4, IB)
    tp = _relayout(tt, patch)
    o5 = _emb_lookup(xv, tp)
    out_t = o5.transpose(0, 1, 3, 2, 4).reshape(NJ, D, NI)
    return out_t.transpose(2, 0, 1)


# speedup vs baseline: 3.9399x; 1.1263x over previous
import functools

import jax
import jax.numpy as jnp
from jax import lax
from jax.experimental import pallas as pl
from jax.experimental.pallas import tpu as pltpu
from jax.experimental.pallas import tpu_sc as plsc

D = 64
NI = 4096
NJ = 200
NW = 32
IB = 128
JB = 8
NTJ = NJ // JB
NTI = NI // IB

_mesh = plsc.VectorSubcoreMesh(core_axis_name="c", subcore_axis_name="s")

NVB = (1000000 // IB)
VB_BASE = NVB // NW
VB_EXTRA = NVB % NW
VBF = IB * D
TFLAT = 1000000 * D
TAIL = (1000000 - NVB * IB) * D


@functools.partial(
    pl.kernel,
    mesh=_mesh,
    out_type=jax.ShapeDtypeStruct((TFLAT,), jnp.float32),
    compiler_params=pltpu.CompilerParams(
        use_tc_tiling_on_sc=True, needs_layout_passes=False),
    scratch_types=[
        pltpu.VMEM((D, IB), jnp.float32),
        pltpu.VMEM((D, IB), jnp.float32),
        pltpu.VMEM((VBF,), jnp.float32),
        pltpu.VMEM((VBF,), jnp.float32),
        pltpu.VMEM((TAIL,), jnp.float32),
        pltpu.SemaphoreType.DMA,
        pltpu.SemaphoreType.DMA,
        pltpu.SemaphoreType.DMA,
        pltpu.SemaphoreType.DMA,
    ],
)
def _relayout(tt_hbm, patch_hbm, tp_hbm, sb0, sb1, db0, db1, pbuf,
              rs0, rs1, ws0, ws1):
    wid = lax.axis_index("s") * 2 + lax.axis_index("c")
    cnt = VB_BASE + jnp.where(wid < VB_EXTRA, 1, 0)
    start = wid * VB_BASE + jnp.minimum(wid, VB_EXTRA)
    sb = (sb0, sb1)
    db = (db0, db1)
    rs = (rs0, rs1)
    ws = (ws0, ws1)
    lanes = lax.iota(jnp.int32, 16)
    vl_g = [lanes + 16 * g for g in range(8)]
    vf_g = [(lanes + 16 * g) * D for g in range(8)]

    def transpose_vb(src, dst):
        @plsc.parallel_loop(0, D, unroll=8)
        def d_body(d0):
            diag = (jnp.full((16,), d0, jnp.int32) + lanes) & (D - 1)
            for g in range(8):
                v = plsc.load_gather(src, [diag, vl_g[g]])
                plsc.store_scatter(dst, [vf_g[g] + diag], v)

    def issue_read(i, h):
        tv = start + i
        return pltpu.async_copy(
            tt_hbm.at[pl.ds(0, D), pl.ds(pl.multiple_of(tv * IB, IB), IB)],
            sb[h], rs[h])

    @pl.when(cnt > 0)
    def _prime():
        issue_read(0, 0)

    def pair_body(t, carry):
        for h in range(2):
            i = t * 2 + h

            @pl.when(i < cnt)
            def _do():
                @pl.when(i + 1 < cnt)
                def _ahead():
                    issue_read(i + 1, 1 - h)
                pltpu.make_async_copy(tt_hbm.at[pl.ds(0, D),
                                                pl.ds(0, IB)],
                                      sb[h], rs[h]).wait()

                @pl.when(i >= 2)
                def _drain():
                    pltpu.make_async_copy(db[h],
                                          tp_hbm.at[pl.ds(0, VBF)],
                                          ws[h]).wait()
                transpose_vb(sb[h], db[h])
                pflat = pl.multiple_of((start + i) * VBF, 8)
                pltpu.async_copy(db[h], tp_hbm.at[pl.ds(pflat, VBF)], ws[h])
        return carry

    lax.fori_loop(0, (VB_BASE + 2) // 2, pair_body, 0)
    for h in range(2):
        @pl.when(cnt > h)
        def _final_drain():
            pltpu.make_async_copy(db[h], tp_hbm.at[pl.ds(0, VBF)],
                                  ws[h]).wait()

    @pl.when(wid == NW - 1)
    def _tail_patch():
        pltpu.sync_copy(patch_hbm, pbuf)
        pltpu.sync_copy(pbuf, tp_hbm.at[pl.ds(TFLAT - TAIL, TAIL)])


@functools.partial(
    pl.kernel,
    mesh=_mesh,
    out_type=jax.ShapeDtypeStruct((NJ, D // 8, NTI, 8, IB), jnp.float32),
    compiler_params=pltpu.CompilerParams(
        use_tc_tiling_on_sc=False, needs_layout_passes=False),
    scratch_types=[
        pltpu.VMEM((JB, IB), jnp.int32),
        pltpu.VMEM((IB, D), jnp.float32),
        pltpu.VMEM((IB, D), jnp.float32),
        pltpu.VMEM((D, IB), jnp.float32),
        pltpu.VMEM((D, IB), jnp.float32),
        pltpu.SemaphoreType.DMA,
        pltpu.SemaphoreType.DMA,
        pltpu.SemaphoreType.DMA,
        pltpu.SemaphoreType.DMA,
    ],
)
def _emb_lookup(xv_hbm, tp_hbm, out_hbm, idx_v,
                rb0, rb1, ob0, ob1, gs0, gs1, ws0, ws1):
    wid = lax.axis_index("s") * 2 + lax.axis_index("c")
    rb = (rb0, rb1)
    ob = (ob0, ob1)
    gs = (gs0, gs1)
    ws = (ws0, ws1)
    lanes = lax.iota(jnp.int32, 16)

    rows_g = [lanes + (16 * g) for g in range(8)]

    def transpose_block(src, dst):
        @plsc.parallel_loop(0, D, unroll=8)
        def d_body(d):
            diag = (jnp.full((16,), d, jnp.int32) + lanes) & (D - 1)
            for g in range(8):
                v = plsc.load_gather(src, [rows_g[g], diag])
                plsc.store_scatter(dst, [diag, rows_g[g]], v)

    def tile_body(tj, carry):
        pltpu.sync_copy(xv_hbm.at[tj, wid], idx_v)
        copies = [None, None]
        writes = [None, None]
        copies[0] = pltpu.async_copy(tp_hbm.at[idx_v.at[0]], rb[0], gs[0])
        for jj in range(JB):
            cur = jj % 2
            nxt = 1 - cur
            if jj + 1 < JB:
                copies[nxt] = pltpu.async_copy(
                    tp_hbm.at[idx_v.at[jj + 1]], rb[nxt], gs[nxt])
            copies[cur].wait()
            if jj >= 2:
                for w in writes[cur]:
                    w.wait()
            transpose_block(rb[cur], ob[cur])
            jabs = tj * JB + jj
            writes[cur] = [
                pltpu.async_copy(
                    ob[cur].at[pl.ds(8 * tk, 8)],
                    out_hbm.at[jabs, tk, wid],
                    ws[cur])
                for tk in range(D // 8)
            ]
        for wl in writes:
            for w in wl:
                w.wait()
        return carry

    lax.fori_loop(0, NTJ, tile_body, 0)


def kernel(x, table):
    xv = x.T.reshape(NTJ, JB, NTI, IB).transpose(0, 2, 1, 3)
    tt = table.T
    patch = table[NVB * IB:].reshape(-1)
    tp = _relayout(tt, patch).reshape(-1, D)
    o5 = _emb_lookup(xv, tp)
    out_t = o5.transpose(0, 1, 3, 2, 4).reshape(NJ, D, NI)
    return out_t.transpose(2, 0, 1)


# speedup vs baseline: 4.1273x; 1.0476x over previous
import functools

import jax
import jax.numpy as jnp
from jax import lax
from jax.experimental import pallas as pl
from jax.experimental.pallas import tpu as pltpu
from jax.experimental.pallas import tpu_sc as plsc

D = 64
NI = 4096
NJ = 200
NW = 32
IB = 128
JB = 8
NTJ = NJ // JB
NTI = NI // IB

_mesh = plsc.VectorSubcoreMesh(core_axis_name="c", subcore_axis_name="s")

NVB = (1000000 // IB)
VB_BASE = NVB // NW
VB_EXTRA = NVB % NW
VBF = IB * D
TFLAT = 1000000 * D
TAIL = (1000000 - NVB * IB) * D


@functools.partial(
    pl.kernel,
    mesh=_mesh,
    out_type=jax.ShapeDtypeStruct((TFLAT,), jnp.float32),
    compiler_params=pltpu.CompilerParams(
        use_tc_tiling_on_sc=True, needs_layout_passes=False),
    scratch_types=[
        pltpu.VMEM((D, IB), jnp.float32),
        pltpu.VMEM((D, IB), jnp.float32),
        pltpu.VMEM((VBF,), jnp.float32),
        pltpu.VMEM((VBF,), jnp.float32),
        pltpu.VMEM((TAIL,), jnp.float32),
        pltpu.SemaphoreType.DMA,
        pltpu.SemaphoreType.DMA,
        pltpu.SemaphoreType.DMA,
        pltpu.SemaphoreType.DMA,
    ],
)
def _relayout(tt_hbm, patch_hbm, tp_hbm, sb0, sb1, db0, db1, pbuf,
              rs0, rs1, ws0, ws1):
    wid = lax.axis_index("s") * 2 + lax.axis_index("c")
    cnt = VB_BASE + jnp.where(wid < VB_EXTRA, 1, 0)
    start = wid * VB_BASE + jnp.minimum(wid, VB_EXTRA)
    sb = (sb0, sb1)
    db = (db0, db1)
    rs = (rs0, rs1)
    ws = (ws0, ws1)
    lanes = lax.iota(jnp.int32, 16)
    vl_g = [lanes + 16 * g for g in range(8)]
    vf_g = [(lanes + 16 * g) * D for g in range(8)]

    def transpose_vb(src, dst):
        @plsc.parallel_loop(0, D, unroll=8)
        def d_body(d0):
            diag = (jnp.full((16,), d0, jnp.int32) + lanes) & (D - 1)
            for g in range(8):
                v = plsc.load_gather(src, [diag, vl_g[g]])
                plsc.store_scatter(dst, [vf_g[g] + diag], v)

    def issue_read(i, h):
        tv = start + i
        return pltpu.async_copy(
            tt_hbm.at[pl.ds(0, D), pl.ds(pl.multiple_of(tv * IB, IB), IB)],
            sb[h], rs[h])

    @pl.when(cnt > 0)
    def _prime():
        issue_read(0, 0)

    def pair_body(t, carry):
        for h in range(2):
            i = t * 2 + h

            @pl.when(i < cnt)
            def _do():
                @pl.when(i + 1 < cnt)
                def _ahead():
                    issue_read(i + 1, 1 - h)
                pltpu.make_async_copy(tt_hbm.at[pl.ds(0, D),
                                                pl.ds(0, IB)],
                                      sb[h], rs[h]).wait()

                @pl.when(i >= 2)
                def _drain():
                    pltpu.make_async_copy(db[h],
                                          tp_hbm.at[pl.ds(0, VBF)],
                                          ws[h]).wait()
                transpose_vb(sb[h], db[h])
                pflat = pl.multiple_of((start + i) * VBF, 8)
                pltpu.async_copy(db[h], tp_hbm.at[pl.ds(pflat, VBF)], ws[h])
        return carry

    lax.fori_loop(0, (VB_BASE + 2) // 2, pair_body, 0)
    for h in range(2):
        @pl.when(cnt > h)
        def _final_drain():
            pltpu.make_async_copy(db[h], tp_hbm.at[pl.ds(0, VBF)],
                                  ws[h]).wait()

    @pl.when(wid == NW - 1)
    def _tail_patch():
        pltpu.sync_copy(patch_hbm, pbuf)
        pltpu.sync_copy(pbuf, tp_hbm.at[pl.ds(TFLAT - TAIL, TAIL)])


@functools.partial(
    pl.kernel,
    mesh=_mesh,
    out_type=jax.ShapeDtypeStruct((NJ, D // 8, NTI, 8, IB), jnp.float32),
    compiler_params=pltpu.CompilerParams(
        use_tc_tiling_on_sc=False, needs_layout_passes=False),
    scratch_types=(
        [pltpu.VMEM((JB, IB), jnp.int32)] * 2
        + [pltpu.VMEM((IB, D), jnp.float32)] * 4
        + [pltpu.VMEM((D, IB), jnp.float32)] * 2
        + [pltpu.SemaphoreType.DMA] * 2
        + [pltpu.SemaphoreType.DMA] * 4
        + [pltpu.SemaphoreType.DMA] * 2
    ),
)
def _emb_lookup(xv_hbm, tp_hbm, out_hbm, ix0, ix1,
                rb0, rb1, rb2, rb3, ob0, ob1,
                is0, is1, gs0, gs1, gs2, gs3, ws0, ws1):
    wid = lax.axis_index("s") * 2 + lax.axis_index("c")
    ix = (ix0, ix1)
    isem = (is0, is1)
    rb = (rb0, rb1, rb2, rb3)
    ob = (ob0, ob1)
    gs = (gs0, gs1, gs2, gs3)
    ws = (ws0, ws1)
    lanes = lax.iota(jnp.int32, 16)

    rows_g = [lanes + (16 * g) for g in range(8)]

    def transpose_block(src, dst):
        @plsc.parallel_loop(0, D, unroll=8)
        def d_body(d):
            diag = (jnp.full((16,), d, jnp.int32) + lanes) & (D - 1)
            for g in range(8):
                v = plsc.load_gather(src, [rows_g[g], diag])
                plsc.store_scatter(dst, [diag, rows_g[g]], v)

    pltpu.async_copy(xv_hbm.at[0, wid], ix[0], isem[0])
    pltpu.async_copy(xv_hbm.at[1, wid], ix[1], isem[1])

    def pair_body(t, carry):
        for h in range(2):
            tj = t * 2 + h

            @pl.when(tj < NTJ)
            def _tile():
                idx_v = ix[h]
                pltpu.make_async_copy(
                    xv_hbm.at[0, wid], idx_v, isem[h]).wait()
                copies = [None] * 4
                writes = [None, None]
                for jj in range(3):
                    copies[jj] = pltpu.async_copy(
                        tp_hbm.at[idx_v.at[jj]], rb[jj], gs[jj])
                for jj in range(JB):
                    s = jj % 4
                    cur = jj % 2
                    copies[s].wait()
                    if jj >= 2:
                        for w in writes[cur]:
                            w.wait()
                    transpose_block(rb[s], ob[cur])
                    if jj + 3 < JB:
                        ns = (jj + 3) % 4
                        copies[ns] = pltpu.async_copy(
                            tp_hbm.at[idx_v.at[jj + 3]], rb[ns], gs[ns])
                    jabs = tj * JB + jj
                    writes[cur] = [
                        pltpu.async_copy(
                            ob[cur].at[pl.ds(8 * tk, 8)],
                            out_hbm.at[jabs, tk, wid],
                            ws[cur])
                        for tk in range(D // 8)
                    ]
                for wl in writes:
                    for w in wl:
                        w.wait()

                @pl.when(tj + 2 < NTJ)
                def _prefetch():
                    pltpu.async_copy(xv_hbm.at[tj + 2, wid], ix[h], isem[h])
        return carry

    lax.fori_loop(0, (NTJ + 2) // 2, pair_body, 0)


def kernel(x, table):
    xv = x.T.reshape(NTJ, JB, NTI, IB).transpose(0, 2, 1, 3)
    tt = table.T
    patch = table[NVB * IB:].reshape(-1)
    tp = _relayout(tt, patch).reshape(-1, D)
    o5 = _emb_lookup(xv, tp)
    out_t = o5.transpose(0, 1, 3, 2, 4).reshape(NJ, D, NI)
    return out_t.transpose(2, 0, 1)


# speedup vs baseline: 4.5765x; 1.1088x over previous
import functools

import jax
import jax.numpy as jnp
from jax import lax
from jax.experimental import pallas as pl
from jax.experimental.pallas import tpu as pltpu
from jax.experimental.pallas import tpu_sc as plsc

D = 64
NI = 4096
NJ = 200
NW = 32
IB = 128
JB = 8
NTJ = NJ // JB
NTI = NI // IB

_mesh = plsc.VectorSubcoreMesh(core_axis_name="c", subcore_axis_name="s")

NVB = (1000000 // IB)
VB_BASE = NVB // NW
VB_EXTRA = NVB % NW
VBF = IB * D
TFLAT = 1000000 * D
TAIL = (1000000 - NVB * IB) * D


@functools.partial(
    pl.kernel,
    mesh=_mesh,
    out_type=jax.ShapeDtypeStruct((TFLAT,), jnp.float32),
    compiler_params=pltpu.CompilerParams(
        use_tc_tiling_on_sc=True, needs_layout_passes=False),
    scratch_types=(
        [pltpu.VMEM((D, IB), jnp.float32)] * 4
        + [pltpu.VMEM((VBF,), jnp.float32)] * 4
        + [pltpu.VMEM((TAIL,), jnp.float32)]
        + [pltpu.SemaphoreType.DMA] * 8
    ),
)
def _relayout(tt_hbm, patch_hbm, tp_hbm, sb0, sb1, sb2, sb3,
              db0, db1, db2, db3, pbuf,
              rs0, rs1, rs2, rs3, ws0, ws1, ws2, ws3):
    wid = lax.axis_index("s") * 2 + lax.axis_index("c")
    cnt = VB_BASE + jnp.where(wid < VB_EXTRA, 1, 0)
    start = wid * VB_BASE + jnp.minimum(wid, VB_EXTRA)
    sb = (sb0, sb1, sb2, sb3)
    db = (db0, db1, db2, db3)
    rs = (rs0, rs1, rs2, rs3)
    ws = (ws0, ws1, ws2, ws3)
    lanes = lax.iota(jnp.int32, 16)
    vl_g = [lanes + 16 * g for g in range(8)]
    vf_g = [(lanes + 16 * g) * D for g in range(8)]

    def transpose_vb(src, dst):
        @plsc.parallel_loop(0, D, unroll=8)
        def d_body(d0):
            diag = (jnp.full((16,), d0, jnp.int32) + lanes) & (D - 1)
            for g in range(8):
                v = plsc.load_gather(src, [diag, vl_g[g]])
                plsc.store_scatter(dst, [vf_g[g] + diag], v)

    def issue_read(i, h):
        tv = start + i
        return pltpu.async_copy(
            tt_hbm.at[pl.ds(0, D), pl.ds(pl.multiple_of(tv * IB, IB), IB)],
            sb[h], rs[h])

    for p in range(3):
        issue_read(p, p)

    def quad_body(t, carry):
        for h in range(4):
            i = t * 4 + h

            @pl.when(i < cnt)
            def _do():
                pltpu.make_async_copy(tt_hbm.at[pl.ds(0, D),
                                                pl.ds(0, IB)],
                                      sb[h], rs[h]).wait()

                @pl.when(i + 3 < cnt)
                def _ahead():
                    issue_read(i + 3, (h + 3) % 4)

                @pl.when(i >= 4)
                def _drain():
                    pltpu.make_async_copy(db[h],
                                          tp_hbm.at[pl.ds(0, VBF)],
                                          ws[h]).wait()
                transpose_vb(sb[h], db[h])
                pflat = pl.multiple_of((start + i) * VBF, 8)
                pltpu.async_copy(db[h], tp_hbm.at[pl.ds(pflat, VBF)], ws[h])
        return carry

    lax.fori_loop(0, (VB_BASE + 4) // 4, quad_body, 0)
    for h in range(4):
        @pl.when(cnt > h)
        def _final_drain():
            pltpu.make_async_copy(db[h], tp_hbm.at[pl.ds(0, VBF)],
                                  ws[h]).wait()

    @pl.when(wid == NW - 1)
    def _tail_patch():
        pltpu.sync_copy(patch_hbm, pbuf)
        pltpu.sync_copy(pbuf, tp_hbm.at[pl.ds(TFLAT - TAIL, TAIL)])


@functools.partial(
    pl.kernel,
    mesh=_mesh,
    out_type=jax.ShapeDtypeStruct((NJ, D // 8, NTI, 8, IB), jnp.float32),
    compiler_params=pltpu.CompilerParams(
        use_tc_tiling_on_sc=False, needs_layout_passes=False),
    scratch_types=(
        [pltpu.VMEM((JB, IB), jnp.int32)] * 2
        + [pltpu.VMEM((IB, D), jnp.float32)] * 4
        + [pltpu.VMEM((D, IB), jnp.float32)] * 2
        + [pltpu.SemaphoreType.DMA] * 2
        + [pltpu.SemaphoreType.DMA] * 4
        + [pltpu.SemaphoreType.DMA] * 2
    ),
)
def _emb_lookup(xv_hbm, tp_hbm, out_hbm, ix0, ix1,
                rb0, rb1, rb2, rb3, ob0, ob1,
                is0, is1, gs0, gs1, gs2, gs3, ws0, ws1):
    wid = lax.axis_index("s") * 2 + lax.axis_index("c")
    ix = (ix0, ix1)
    isem = (is0, is1)
    rb = (rb0, rb1, rb2, rb3)
    ob = (ob0, ob1)
    gs = (gs0, gs1, gs2, gs3)
    ws = (ws0, ws1)
    lanes = lax.iota(jnp.int32, 16)

    rows_g = [lanes + (16 * g) for g in range(8)]

    def transpose_block(src, dst):
        @plsc.parallel_loop(0, D, unroll=8)
        def d_body(d):
            diag = (jnp.full((16,), d, jnp.int32) + lanes) & (D - 1)
            for g in range(8):
                v = plsc.load_gather(src, [rows_g[g], diag])
                plsc.store_scatter(dst, [diag, rows_g[g]], v)

    pltpu.async_copy(xv_hbm.at[0, wid], ix[0], isem[0])
    pltpu.async_copy(xv_hbm.at[1, wid], ix[1], isem[1])

    def pair_body(t, carry):
        for h in range(2):
            tj = t * 2 + h

            @pl.when(tj < NTJ)
            def _tile():
                idx_v = ix[h]
                pltpu.make_async_copy(
                    xv_hbm.at[0, wid], idx_v, isem[h]).wait()
                copies = [None] * 4
                writes = [None, None]
                for jj in range(3):
                    copies[jj] = pltpu.async_copy(
                        tp_hbm.at[idx_v.at[jj]], rb[jj], gs[jj])
                for jj in range(JB):
                    s = jj % 4
                    cur = jj % 2
                    copies[s].wait()
                    if jj >= 2:
                        for w in writes[cur]:
                            w.wait()
                    transpose_block(rb[s], ob[cur])
                    if jj + 3 < JB:
                        ns = (jj + 3) % 4
                        copies[ns] = pltpu.async_copy(
                            tp_hbm.at[idx_v.at[jj + 3]], rb[ns], gs[ns])
                    jabs = tj * JB + jj
                    writes[cur] = [
                        pltpu.async_copy(
                            ob[cur].at[pl.ds(8 * tk, 8)],
                            out_hbm.at[jabs, tk, wid],
                            ws[cur])
                        for tk in range(D // 8)
                    ]
                for wl in writes:
                    for w in wl:
                        w.wait()

                @pl.when(tj + 2 < NTJ)
                def _prefetch():
                    pltpu.async_copy(xv_hbm.at[tj + 2, wid], ix[h], isem[h])
        return carry

    lax.fori_loop(0, (NTJ + 2) // 2, pair_body, 0)


def kernel(x, table):
    xv = x.T.reshape(NTJ, JB, NTI, IB).transpose(0, 2, 1, 3)
    tt = table.T
    patch = table[NVB * IB:].reshape(-1)
    tp = _relayout(tt, patch).reshape(-1, D)
    o5 = _emb_lookup(xv, tp)
    out_t = o5.transpose(0, 1, 3, 2, 4).reshape(NJ, D, NI)
    return out_t.transpose(2, 0, 1)
